# Initial kernel scaffold; baseline (speedup 1.0000x reference)
import jax
import jax.numpy as jnp
from jax.experimental import pallas as pl


def kernel(batch_pred, batch_group):
    raise NotImplementedError("write your pallas kernel here")


# speedup vs baseline: 2.0619x; 2.0619x over previous
import jax
import jax.numpy as jnp
from jax.experimental import pallas as pl
from jax.experimental.pallas import tpu as pltpu

N = 16384
R = 128
C = 128


def _wdloss_body(v_ref, g_ref, out_ref):
    v = v_ref[...]
    g = g_ref[...]
    inf = jnp.float32(jnp.inf)
    a0 = jnp.where(g == 0, v, inf)
    a1 = jnp.where(g == 1, v, inf)
    a = jnp.concatenate([a0, a1], axis=0)

    row = jax.lax.broadcasted_iota(jnp.int32, (2 * R, C), 0) % R
    col = jax.lax.broadcasted_iota(jnp.int32, (2 * R, C), 1)
    flat = row * C + col

    for k_log in range(1, 15):
        k = 1 << k_log
        up = (flat & k) == 0
        for j_log in range(k_log - 1, -1, -1):
            j = 1 << j_log
            if j < C:
                pu = jnp.concatenate([a[:, j:], a[:, :j]], axis=1)
                pd = jnp.concatenate([a[:, C - j:], a[:, : C - j]], axis=1)
            else:
                jr = j // C
                pu = jnp.concatenate([a[jr:], a[:jr]], axis=0)
                pd = jnp.concatenate([a[2 * R - jr:], a[: 2 * R - jr]], axis=0)
            lower = (flat & j) == 0
            partner = jnp.where(lower, pu, pd)
            take_min = lower == up
            a = jnp.where(take_min, jnp.minimum(a, partner),
                          jnp.maximum(a, partner))

    g0s = a[:R, :]
    g1s = a[R:, :]
    n0 = jnp.sum(jnp.where(g == 0, 1, 0))
    m = jnp.minimum(n0, N - n0)
    valid = flat[:R, :] < m
    diffs = jnp.where(valid, jnp.abs(g0s - g1s), jnp.float32(0.0))
    out_ref[0, 0] = jnp.sum(diffs) / m.astype(jnp.float32)


def kernel(batch_pred, batch_group):
    v = batch_pred.reshape(R, C)
    g = batch_group.astype(jnp.int32).reshape(R, C)
    out = pl.pallas_call(
        _wdloss_body,
        out_shape=jax.ShapeDtypeStruct((1, 1), jnp.float32),
        out_specs=pl.BlockSpec(memory_space=pltpu.SMEM),
    )(v, g)
    return out[0, 0]
